# baseline (speedup 1.0000x reference)
import functools

import jax
import jax.numpy as jnp
from jax import lax
from jax.experimental import pallas as pl
from jax.experimental.pallas import tpu as pltpu
from jax.experimental.pallas import tpu_sc as plsc

N_NODES = 10000
N_EDGES = 320000
N_LAYERS = 3
CHANNELS = 128

NC = 2
NS = 16
NW = NC * NS
EPW = N_EDGES // NW
CHUNK = 80
CPW = EPW // CHUNK
NPAD = 10240
NPS = NPAD // NS


def _sc_body(layer, x_hbm, el_hbm, z_hbm, out_hbm,
             src_v, dst_v, rows_a, rows_b, acc, sem_a, sem_b):
    c = lax.axis_index("c")
    s = lax.axis_index("s")
    wid = c * NS + s
    src_base = (2 * layer + 0) * N_EDGES + wid * EPW
    dst_base = (2 * layer + 1) * N_EDGES + wid * EPW

    def gather(i, rows, sem):
        off = pl.multiple_of(i * CHUNK, CHUNK)
        pltpu.async_copy(x_hbm.at[src_v.at[pl.ds(off, CHUNK)]], rows, sem)

    def gather_wait(i, rows, sem):
        off = pl.multiple_of(i * CHUNK, CHUNK)
        pltpu.make_async_copy(x_hbm.at[src_v.at[pl.ds(off, CHUNK)]],
                              rows, sem).wait()

    def scatter(i, rows):
        off = pl.multiple_of(i * CHUNK, CHUNK)
        pltpu.sync_copy(rows, acc.at[dst_v.at[pl.ds(off, CHUNK)]], add=True)

    pltpu.sync_copy(z_hbm, acc.at[pl.ds(s * NPS, NPS)])
    pltpu.sync_copy(el_hbm.at[pl.ds(src_base, EPW)], src_v)
    pltpu.sync_copy(el_hbm.at[pl.ds(dst_base, EPW)], dst_v)
    gather(0, rows_a, sem_a)
    plsc.subcore_barrier()

    def loop(j, carry):
        i0 = 2 * j
        gather(i0 + 1, rows_b, sem_b)
        gather_wait(i0, rows_a, sem_a)
        scatter(i0, rows_a)
        gather(i0 + 2, rows_a, sem_a)
        gather_wait(i0 + 1, rows_b, sem_b)
        scatter(i0 + 1, rows_b)
        return carry

    lax.fori_loop(0, (CPW - 1) // 2, loop, 0)
    gather_wait(CPW - 1, rows_a, sem_a)
    scatter(CPW - 1, rows_a)

    plsc.subcore_barrier()
    pltpu.sync_copy(acc.at[pl.ds(s * NPS, NPS)],
                    out_hbm.at[c, pl.ds(s * NPS, NPS)])


def _make_sc_segment_sum(layer):
    mesh = plsc.VectorSubcoreMesh(core_axis_name="c", subcore_axis_name="s",
                                  num_cores=NC, num_subcores=NS)
    return jax.jit(pl.kernel(
        functools.partial(_sc_body, layer),
        out_type=jax.ShapeDtypeStruct((NC, NPAD, CHANNELS), jnp.float32),
        mesh=mesh,
        scratch_types=[
            pltpu.VMEM((EPW,), jnp.int32),
            pltpu.VMEM((EPW,), jnp.int32),
            pltpu.VMEM((CHUNK, CHANNELS), jnp.float32),
            pltpu.VMEM((CHUNK, CHANNELS), jnp.float32),
            pltpu.VMEM_SHARED((NPAD, CHANNELS), jnp.float32),
            pltpu.SemaphoreType.DMA,
            pltpu.SemaphoreType.DMA,
        ],
    ))


_sc_segment_sum = [_make_sc_segment_sum(l) for l in range(N_LAYERS)]


BM = 2000


def _tc_mid_body(x_ref, a_ref, wr_ref, wn_ref, b_ref, o_ref):
    a = a_ref[0] + a_ref[1]
    y = jnp.dot(x_ref[...], wr_ref[...], preferred_element_type=jnp.float32)
    y = y + jnp.dot(a, wn_ref[...], preferred_element_type=jnp.float32)
    y = y + b_ref[...]
    o_ref[...] = jnp.where(y >= 0, y, 0.01 * y)


def _tc_final_body(x_ref, a_ref, wr_ref, wn_ref, b_ref, wo_ref, bo_ref, o_ref):
    a = a_ref[0] + a_ref[1]
    y = jnp.dot(x_ref[...], wr_ref[...], preferred_element_type=jnp.float32)
    y = y + jnp.dot(a, wn_ref[...], preferred_element_type=jnp.float32)
    y = y + b_ref[...]
    o_ref[...] = (jnp.dot(y, wo_ref[...], preferred_element_type=jnp.float32)
                  + bo_ref[...])


_row_spec = pl.BlockSpec((BM, CHANNELS), lambda i: (i, 0))
_agg_spec = pl.BlockSpec((NC, BM, CHANNELS), lambda i: (0, i, 0))
_w_spec = pl.BlockSpec((CHANNELS, CHANNELS), lambda i: (0, 0))
_b_spec = pl.BlockSpec((1, CHANNELS), lambda i: (0, 0))


@jax.jit
def _tc_mid(x, agg, wr, wn, bb):
    return pl.pallas_call(
        _tc_mid_body,
        grid=(N_NODES // BM,),
        in_specs=[_row_spec, _agg_spec, _w_spec, _w_spec, _b_spec],
        out_specs=_row_spec,
        out_shape=jax.ShapeDtypeStruct((N_NODES, CHANNELS), jnp.float32),
    )(x, agg, wr, wn, bb)


@jax.jit
def _tc_final(x, agg, wr, wn, bb, wo, bo):
    return pl.pallas_call(
        _tc_final_body,
        grid=(N_NODES // BM,),
        in_specs=[_row_spec, _agg_spec, _w_spec, _w_spec, _b_spec,
                  _w_spec, _b_spec],
        out_specs=_row_spec,
        out_shape=jax.ShapeDtypeStruct((N_NODES, CHANNELS), jnp.float32),
    )(x, agg, wr, wn, bb, wo, bo)


def kernel(z, edge_list, W_root, W_nei, b, W_out, b_out):
    el_flat = edge_list.astype(jnp.int32).reshape(-1)
    zeros = jnp.zeros((NPS, CHANNELS), jnp.float32)
    x = z
    for i in range(N_LAYERS):
        lidx = N_LAYERS - 1 - i
        agg = _sc_segment_sum[lidx](x, el_flat, zeros)
        if i < N_LAYERS - 1:
            x = _tc_mid(x, agg, W_root[i], W_nei[i], b[i].reshape(1, CHANNELS))
        else:
            x = _tc_final(x, agg, W_root[i], W_nei[i], b[i].reshape(1, CHANNELS),
                          W_out, b_out.reshape(1, CHANNELS))
    return x


# speedup vs baseline: 1.0922x; 1.0922x over previous
import functools

import jax
import jax.numpy as jnp
from jax import lax
from jax.experimental import pallas as pl
from jax.experimental.pallas import tpu as pltpu
from jax.experimental.pallas import tpu_sc as plsc

N_NODES = 10000
N_EDGES = 320000
N_LAYERS = 3
CHANNELS = 128

NC = 2
NS = 16
NW = NC * NS
EPW = N_EDGES // NW
CHUNK = 125
CPW = EPW // CHUNK
WIN = 8
NPAD = 10240
NPS = NPAD // NS


def _sc_body(x_hbm, el_hbm, z_hbm, out_hbm,
             win_sa, win_da, win_sb, win_db, rows_a, rows_b, acc,
             sem_a, sem_b, sem_wa, sem_wb):
    c = lax.axis_index("c")
    s = lax.axis_index("s")
    wid = c * NS + s

    def refill(win_s, win_d, w, sem):
        pltpu.async_copy(el_hbm.at[0, wid, pl.ds(w * WIN, WIN)], win_s, sem)
        pltpu.async_copy(el_hbm.at[1, wid, pl.ds(w * WIN, WIN)], win_d, sem)

    def wait_refill(win_s, win_d, w, sem):
        pltpu.make_async_copy(el_hbm.at[0, wid, pl.ds(w * WIN, WIN)],
                              win_s, sem).wait()
        pltpu.make_async_copy(el_hbm.at[1, wid, pl.ds(w * WIN, WIN)],
                              win_d, sem).wait()

    def rows_of(i):
        return (rows_a, sem_a) if i % 2 == 0 else (rows_b, sem_b)

    pltpu.sync_copy(z_hbm, acc.at[pl.ds(s * NPS, NPS)])
    pltpu.sync_copy(el_hbm.at[0, wid, pl.ds(0, WIN)], win_sa)
    pltpu.sync_copy(el_hbm.at[1, wid, pl.ds(0, WIN)], win_da)
    pltpu.async_copy(x_hbm.at[win_sa.at[0]], rows_a, sem_a)
    plsc.subcore_barrier()

    def step(t, base, last):
        cw_s, cw_d = (win_sa, win_da) if t < 8 else (win_sb, win_db)
        if t == 7:
            wait_refill(win_sb, win_db, base // WIN + 1, sem_wb)
        if not last:
            if t == 15:
                wait_refill(win_sa, win_da, base // WIN + 2, sem_wa)
        if not (last and t == 15):
            nw_s = win_sa if ((t + 1) % 16) < 8 else win_sb
            nrows, nsem = rows_of(t + 1)
            pltpu.async_copy(x_hbm.at[nw_s.at[(t + 1) % 8]], nrows, nsem)
        if t == 8 and not last:
            refill(win_sa, win_da, base // WIN + 2, sem_wa)
        rows, sem = rows_of(t)
        pltpu.make_async_copy(x_hbm.at[cw_s.at[t % 8]], rows, sem).wait()
        pltpu.sync_copy(rows, acc.at[cw_d.at[t % 8]], add=True)

    def superstep(sidx, carry):
        base = 2 * WIN * sidx
        refill(win_sb, win_db, base // WIN + 1, sem_wb)
        for t in range(16):
            step(t, base, last=False)
        return carry

    n_super = CPW // (2 * WIN)
    lax.fori_loop(0, n_super - 1, superstep, 0)
    final_base = 2 * WIN * (n_super - 1)
    refill(win_sb, win_db, final_base // WIN + 1, sem_wb)
    for t in range(16):
        step(t, final_base, last=True)

    plsc.subcore_barrier()
    pltpu.sync_copy(acc.at[pl.ds(s * NPS, NPS)],
                    out_hbm.at[c, pl.ds(s * NPS, NPS)])


def _make_sc_segment_sum():
    mesh = plsc.VectorSubcoreMesh(core_axis_name="c", subcore_axis_name="s",
                                  num_cores=NC, num_subcores=NS)
    return jax.jit(pl.kernel(
        _sc_body,
        out_type=jax.ShapeDtypeStruct((NC, NPAD, CHANNELS), jnp.float32),
        mesh=mesh,
        scratch_types=[
            pltpu.VMEM((WIN, CHUNK), jnp.int32),
            pltpu.VMEM((WIN, CHUNK), jnp.int32),
            pltpu.VMEM((WIN, CHUNK), jnp.int32),
            pltpu.VMEM((WIN, CHUNK), jnp.int32),
            pltpu.VMEM((CHUNK, CHANNELS), jnp.float32),
            pltpu.VMEM((CHUNK, CHANNELS), jnp.float32),
            pltpu.VMEM_SHARED((NPAD, CHANNELS), jnp.float32),
            pltpu.SemaphoreType.DMA,
            pltpu.SemaphoreType.DMA,
            pltpu.SemaphoreType.DMA,
            pltpu.SemaphoreType.DMA,
        ],
    ))


_sc_segment_sum = _make_sc_segment_sum()


BM = 2000


def _tc_mid_body(x_ref, a_ref, wr_ref, wn_ref, b_ref, o_ref):
    a = a_ref[0] + a_ref[1]
    y = jnp.dot(x_ref[...], wr_ref[...], preferred_element_type=jnp.float32)
    y = y + jnp.dot(a, wn_ref[...], preferred_element_type=jnp.float32)
    y = y + b_ref[...]
    o_ref[...] = jnp.where(y >= 0, y, 0.01 * y)


def _tc_final_body(x_ref, a_ref, wr_ref, wn_ref, b_ref, wo_ref, bo_ref, o_ref):
    a = a_ref[0] + a_ref[1]
    y = jnp.dot(x_ref[...], wr_ref[...], preferred_element_type=jnp.float32)
    y = y + jnp.dot(a, wn_ref[...], preferred_element_type=jnp.float32)
    y = y + b_ref[...]
    o_ref[...] = (jnp.dot(y, wo_ref[...], preferred_element_type=jnp.float32)
                  + bo_ref[...])


_row_spec = pl.BlockSpec((BM, CHANNELS), lambda i: (i, 0))
_agg_spec = pl.BlockSpec((NC, BM, CHANNELS), lambda i: (0, i, 0))
_w_spec = pl.BlockSpec((CHANNELS, CHANNELS), lambda i: (0, 0))
_b_spec = pl.BlockSpec((1, CHANNELS), lambda i: (0, 0))


@jax.jit
def _tc_mid(x, agg, wr, wn, bb):
    return pl.pallas_call(
        _tc_mid_body,
        grid=(N_NODES // BM,),
        in_specs=[_row_spec, _agg_spec, _w_spec, _w_spec, _b_spec],
        out_specs=_row_spec,
        out_shape=jax.ShapeDtypeStruct((N_NODES, CHANNELS), jnp.float32),
    )(x, agg, wr, wn, bb)


@jax.jit
def _tc_final(x, agg, wr, wn, bb, wo, bo):
    return pl.pallas_call(
        _tc_final_body,
        grid=(N_NODES // BM,),
        in_specs=[_row_spec, _agg_spec, _w_spec, _w_spec, _b_spec,
                  _w_spec, _b_spec],
        out_specs=_row_spec,
        out_shape=jax.ShapeDtypeStruct((N_NODES, CHANNELS), jnp.float32),
    )(x, agg, wr, wn, bb, wo, bo)


def kernel(z, edge_list, W_root, W_nei, b, W_out, b_out):
    el = edge_list.astype(jnp.int32)
    el_l = [el[lidx].reshape(2, NW, CPW, CHUNK) for lidx in range(N_LAYERS)]
    zeros = jnp.zeros((NPS, CHANNELS), jnp.float32)
    x = z
    for i in range(N_LAYERS):
        lidx = N_LAYERS - 1 - i
        agg = _sc_segment_sum(x, el_l[lidx], zeros)
        if i < N_LAYERS - 1:
            x = _tc_mid(x, agg, W_root[i], W_nei[i], b[i].reshape(1, CHANNELS))
        else:
            x = _tc_final(x, agg, W_root[i], W_nei[i], b[i].reshape(1, CHANNELS),
                          W_out, b_out.reshape(1, CHANNELS))
    return x


# speedup vs baseline: 1.1185x; 1.0241x over previous
import functools

import jax
import jax.numpy as jnp
from jax import lax
from jax.experimental import pallas as pl
from jax.experimental.pallas import tpu as pltpu
from jax.experimental.pallas import tpu_sc as plsc

N_NODES = 10000
N_EDGES = 320000
N_LAYERS = 3
CHANNELS = 128

NC = 2
NS = 16
NW = NC * NS
EPW = N_EDGES // NW
CHUNK = 125
CPW = EPW // CHUNK
WIN = 8
NPAD = 10240
NPS = NPAD // NS


def _sc_body(x_hbm, el_hbm, out_hbm,
             win_sa, win_da, win_sb, win_db, rows_a, rows_b, acc,
             sem_a, sem_b, sem_wa, sem_wb):
    c = lax.axis_index("c")
    s = lax.axis_index("s")
    wid = c * NS + s

    def refill(win_s, win_d, w, sem):
        pltpu.async_copy(el_hbm.at[0, wid, pl.ds(w * WIN, WIN)], win_s, sem)
        pltpu.async_copy(el_hbm.at[1, wid, pl.ds(w * WIN, WIN)], win_d, sem)

    def wait_refill(win_s, win_d, w, sem):
        pltpu.make_async_copy(el_hbm.at[0, wid, pl.ds(w * WIN, WIN)],
                              win_s, sem).wait()
        pltpu.make_async_copy(el_hbm.at[1, wid, pl.ds(w * WIN, WIN)],
                              win_d, sem).wait()

    def rows_of(i):
        return (rows_a, sem_a) if i % 2 == 0 else (rows_b, sem_b)

    zv = jnp.zeros((16,), jnp.float32)

    def zrow(i, carry):
        for g in range(CHANNELS // 16):
            rows_a[i, pl.ds(g * 16, 16)] = zv
        return carry

    lax.fori_loop(0, 64, zrow, 0)
    for r in range(NPS // 64):
        pltpu.sync_copy(rows_a.at[pl.ds(0, 64)],
                        acc.at[pl.ds(s * NPS + r * 64, 64)])
    pltpu.sync_copy(el_hbm.at[0, wid, pl.ds(0, WIN)], win_sa)
    pltpu.sync_copy(el_hbm.at[1, wid, pl.ds(0, WIN)], win_da)
    pltpu.async_copy(x_hbm.at[win_sa.at[0]], rows_a, sem_a)
    plsc.subcore_barrier()

    def step(t, base, last):
        cw_s, cw_d = (win_sa, win_da) if t < 8 else (win_sb, win_db)
        if t == 7:
            wait_refill(win_sb, win_db, base // WIN + 1, sem_wb)
        if not last:
            if t == 15:
                wait_refill(win_sa, win_da, base // WIN + 2, sem_wa)
        if not (last and t == 15):
            nw_s = win_sa if ((t + 1) % 16) < 8 else win_sb
            nrows, nsem = rows_of(t + 1)
            pltpu.async_copy(x_hbm.at[nw_s.at[(t + 1) % 8]], nrows, nsem)
        if t == 8 and not last:
            refill(win_sa, win_da, base // WIN + 2, sem_wa)
        rows, sem = rows_of(t)
        pltpu.make_async_copy(x_hbm.at[cw_s.at[t % 8]], rows, sem).wait()
        pltpu.sync_copy(rows, acc.at[cw_d.at[t % 8]], add=True)

    def superstep(sidx, carry):
        base = 2 * WIN * sidx
        refill(win_sb, win_db, base // WIN + 1, sem_wb)
        for t in range(16):
            step(t, base, last=False)
        return carry

    n_super = CPW // (2 * WIN)
    lax.fori_loop(0, n_super - 1, superstep, 0)
    final_base = 2 * WIN * (n_super - 1)
    refill(win_sb, win_db, final_base // WIN + 1, sem_wb)
    for t in range(16):
        step(t, final_base, last=True)

    plsc.subcore_barrier()
    pltpu.sync_copy(acc.at[pl.ds(s * NPS, NPS)],
                    out_hbm.at[c, pl.ds(s * NPS, NPS)])


def _make_sc_segment_sum():
    mesh = plsc.VectorSubcoreMesh(core_axis_name="c", subcore_axis_name="s",
                                  num_cores=NC, num_subcores=NS)
    return jax.jit(pl.kernel(
        _sc_body,
        out_type=jax.ShapeDtypeStruct((NC, NPAD, CHANNELS), jnp.float32),
        mesh=mesh,
        scratch_types=[
            pltpu.VMEM((WIN, CHUNK), jnp.int32),
            pltpu.VMEM((WIN, CHUNK), jnp.int32),
            pltpu.VMEM((WIN, CHUNK), jnp.int32),
            pltpu.VMEM((WIN, CHUNK), jnp.int32),
            pltpu.VMEM((CHUNK, CHANNELS), jnp.float32),
            pltpu.VMEM((CHUNK, CHANNELS), jnp.float32),
            pltpu.VMEM_SHARED((NPAD, CHANNELS), jnp.float32),
            pltpu.SemaphoreType.DMA,
            pltpu.SemaphoreType.DMA,
            pltpu.SemaphoreType.DMA,
            pltpu.SemaphoreType.DMA,
        ],
    ))


_sc_segment_sum = _make_sc_segment_sum()


BM = 2000


def _tc_mid_body(x_ref, a_ref, wr_ref, wn_ref, b_ref, o_ref):
    a = a_ref[0] + a_ref[1]
    y = jnp.dot(x_ref[...], wr_ref[...], preferred_element_type=jnp.float32)
    y = y + jnp.dot(a, wn_ref[...], preferred_element_type=jnp.float32)
    y = y + b_ref[...]
    o_ref[...] = jnp.where(y >= 0, y, 0.01 * y)


def _tc_final_body(x_ref, a_ref, wr_ref, wn_ref, b_ref, wo_ref, bo_ref, o_ref):
    a = a_ref[0] + a_ref[1]
    y = jnp.dot(x_ref[...], wr_ref[...], preferred_element_type=jnp.float32)
    y = y + jnp.dot(a, wn_ref[...], preferred_element_type=jnp.float32)
    y = y + b_ref[...]
    o_ref[...] = (jnp.dot(y, wo_ref[...], preferred_element_type=jnp.float32)
                  + bo_ref[...])


_row_spec = pl.BlockSpec((BM, CHANNELS), lambda i: (i, 0))
_agg_spec = pl.BlockSpec((NC, BM, CHANNELS), lambda i: (0, i, 0))
_w_spec = pl.BlockSpec((CHANNELS, CHANNELS), lambda i: (0, 0))
_b_spec = pl.BlockSpec((1, CHANNELS), lambda i: (0, 0))


@jax.jit
def _tc_mid(x, agg, wr, wn, bb):
    return pl.pallas_call(
        _tc_mid_body,
        grid=(N_NODES // BM,),
        in_specs=[_row_spec, _agg_spec, _w_spec, _w_spec, _b_spec],
        out_specs=_row_spec,
        out_shape=jax.ShapeDtypeStruct((N_NODES, CHANNELS), jnp.float32),
    )(x, agg, wr, wn, bb)


@jax.jit
def _tc_final(x, agg, wr, wn, bb, wo, bo):
    return pl.pallas_call(
        _tc_final_body,
        grid=(N_NODES // BM,),
        in_specs=[_row_spec, _agg_spec, _w_spec, _w_spec, _b_spec,
                  _w_spec, _b_spec],
        out_specs=_row_spec,
        out_shape=jax.ShapeDtypeStruct((N_NODES, CHANNELS), jnp.float32),
    )(x, agg, wr, wn, bb, wo, bo)


def kernel(z, edge_list, W_root, W_nei, b, W_out, b_out):
    el = edge_list.astype(jnp.int32)
    el_l = [el[lidx].reshape(2, NW, CPW, CHUNK) for lidx in range(N_LAYERS)]
    x = z
    for i in range(N_LAYERS):
        lidx = N_LAYERS - 1 - i
        agg = _sc_segment_sum(x, el_l[lidx])
        if i < N_LAYERS - 1:
            x = _tc_mid(x, agg, W_root[i], W_nei[i], b[i].reshape(1, CHANNELS))
        else:
            x = _tc_final(x, agg, W_root[i], W_nei[i], b[i].reshape(1, CHANNELS),
                          W_out, b_out.reshape(1, CHANNELS))
    return x
